# Initial kernel scaffold; baseline (speedup 1.0000x reference)
import jax
import jax.numpy as jnp
from jax.experimental import pallas as pl


def kernel(input, hidden, edge_index, edge_weight, W, b):
    raise NotImplementedError("write your pallas kernel here")


# speedup vs baseline: 3.1119x; 3.1119x over previous
import functools

import jax
import jax.numpy as jnp
from jax import lax
from jax.experimental import pallas as pl
from jax.experimental.pallas import tpu as pltpu
from jax.experimental.pallas import tpu_sc as plsc

N = 10000
E = 320000
D = 128
K = 10

CH = 128
NTILES = 16
CHUNKS = -(-E // CH)
CPT = -(-CHUNKS // NTILES)
E_PAD = CPT * NTILES * CH
NPAD = 10240
NPT = NPAD // NTILES


def _spmv_body(tl_hbm, tr_hbm, z_hbm, src_hbm, dst_hbm, w_hbm,
               ol_hbm, or_hbm,
               acc, idx_v, dst_v, w_v, rows_v, sem):
    c = lax.axis_index("c")
    s = lax.axis_index("s")
    rows0 = s * NPT

    pltpu.sync_copy(z_hbm.at[pl.ds(rows0, NPT)], acc.at[pl.ds(rows0, NPT)])
    plsc.subcore_barrier()

    def chunk_body(jj, carry):
        base = (jj * NTILES + s) * CH
        pltpu.sync_copy(src_hbm.at[pl.ds(base, CH)], idx_v)
        pltpu.sync_copy(dst_hbm.at[pl.ds(base, CH)], dst_v)
        pltpu.sync_copy(w_hbm.at[pl.ds(base, CH)], w_v)

        @pl.when(c == 0)
        def _():
            pltpu.async_copy(tl_hbm.at[idx_v], rows_v, sem).wait()

        @pl.when(c == 1)
        def _():
            pltpu.async_copy(tr_hbm.at[idx_v], rows_v, sem).wait()

        def scale_body(g, carry2):
            wv16 = w_v[pl.ds(g * 16, 16)]
            for e16 in range(16):
                e = g * 16 + e16
                wb = lax.gather(
                    wv16, jnp.full((16, 1), e16, jnp.int32),
                    lax.GatherDimensionNumbers(
                        offset_dims=(), collapsed_slice_dims=(0,),
                        start_index_map=(0,)),
                    slice_sizes=(1,),
                    mode=lax.GatherScatterMode.PROMISE_IN_BOUNDS)
                for v in range(D // 16):
                    rows_v[e, pl.ds(v * 16, 16)] = (
                        rows_v[e, pl.ds(v * 16, 16)] * wb)
            return carry2

        lax.fori_loop(0, CH // 16, scale_body, 0)
        pltpu.sync_copy(rows_v, acc.at[dst_v], add=True)
        return carry

    lax.fori_loop(0, CPT, chunk_body, 0)
    plsc.subcore_barrier()

    @pl.when(c == 0)
    def _():
        pltpu.sync_copy(acc.at[pl.ds(rows0, NPT)], ol_hbm.at[pl.ds(rows0, NPT)])

    @pl.when(c == 1)
    def _():
        pltpu.sync_copy(acc.at[pl.ds(rows0, NPT)], or_hbm.at[pl.ds(rows0, NPT)])


@functools.cache
def _spmv_step():
    return pl.kernel(
        _spmv_body,
        out_type=[jax.ShapeDtypeStruct((NPAD, D), jnp.float32),
                  jax.ShapeDtypeStruct((NPAD, D), jnp.float32)],
        mesh=plsc.VectorSubcoreMesh(core_axis_name="c", subcore_axis_name="s"),
        scratch_types=[
            pltpu.VMEM_SHARED((NPAD, D), jnp.float32),
            pltpu.VMEM((CH,), jnp.int32),
            pltpu.VMEM((CH,), jnp.int32),
            pltpu.VMEM((CH,), jnp.float32),
            pltpu.VMEM((CH, D), jnp.float32),
            pltpu.SemaphoreType.DMA,
        ],
    )


BR = 2000
NP2 = 2 * (K + 1)


def _proj_body(*refs):
    pieces = refs[:NP2]
    w_ref = refs[NP2]
    g_ref = refs[NP2 + 1]
    wb = w_ref[...].astype(jnp.bfloat16)
    acc = jnp.zeros((BR, D), jnp.float32)
    for j in range(NP2):
        acc = acc + jnp.dot(pieces[j][...].astype(jnp.bfloat16), wb[j],
                            preferred_element_type=jnp.float32)
    g_ref[...] = acc


def _project(pieces, W22):
    return pl.pallas_call(
        _proj_body,
        grid=(N // BR,),
        in_specs=[pl.BlockSpec((BR, D), lambda i: (i, 0))] * NP2
        + [pl.BlockSpec((NP2, D, D), lambda i: (0, 0, 0))],
        out_specs=pl.BlockSpec((BR, D), lambda i: (i, 0)),
        out_shape=jax.ShapeDtypeStruct((N, D), jnp.float32),
    )(*pieces, W22)


def _mid_body(t1_ref, h_ref, b_ref, z_ref, rh_ref):
    z = jax.nn.sigmoid(t1_ref[...] + b_ref[0])
    z_ref[...] = z
    rh_ref[...] = z * h_ref[...]


def _mid(t1, hidden, b2):
    return pl.pallas_call(
        _mid_body,
        grid=(N // BR,),
        in_specs=[
            pl.BlockSpec((BR, D), lambda i: (i, 0)),
            pl.BlockSpec((BR, D), lambda i: (i, 0)),
            pl.BlockSpec((1, D), lambda i: (0, 0)),
        ],
        out_specs=[
            pl.BlockSpec((BR, D), lambda i: (i, 0)),
            pl.BlockSpec((BR, D), lambda i: (i, 0)),
        ],
        out_shape=[
            jax.ShapeDtypeStruct((N, D), jnp.float32),
            jax.ShapeDtypeStruct((N, D), jnp.float32),
        ],
    )(t1, hidden, b2)


def _final_body(t2_ref, z_ref, h_ref, b_ref, out_ref):
    cval = jax.nn.sigmoid(t2_ref[...] + b_ref[0])
    z = z_ref[...]
    out_ref[...] = z * h_ref[...] + (1.0 - z) * cval


def _final(t2, z, hidden, b2):
    return pl.pallas_call(
        _final_body,
        grid=(N // BR,),
        in_specs=[
            pl.BlockSpec((BR, D), lambda i: (i, 0)),
            pl.BlockSpec((BR, D), lambda i: (i, 0)),
            pl.BlockSpec((BR, D), lambda i: (i, 0)),
            pl.BlockSpec((1, D), lambda i: (0, 0)),
        ],
        out_specs=pl.BlockSpec((BR, D), lambda i: (i, 0)),
        out_shape=jax.ShapeDtypeStruct((N, D), jnp.float32),
    )(t2, z, hidden, b2)


def _gconv(t0l, t0r, src, dst, w, zeros, W22):
    step = _spmv_step()
    pieces = [t0l, t0r]
    pl_, pr_ = t0l, t0r
    for _ in range(K):
        pl_, pr_ = step(pl_, pr_, zeros, src, dst, w)
        pieces.extend([pl_, pr_])
    return _project(pieces, W22)


def kernel(input, hidden, edge_index, edge_weight, W, b):
    W22 = W.reshape(NP2, D, D)
    b2 = b.reshape(1, D)
    pad = E_PAD - E
    src = jnp.concatenate([edge_index[0], jnp.zeros((pad,), jnp.int32)])
    dst = jnp.concatenate([edge_index[1], jnp.zeros((pad,), jnp.int32)])
    w = jnp.concatenate([edge_weight, jnp.zeros((pad,), jnp.float32)])
    zeros = jnp.zeros((NPAD, D), jnp.float32)

    xp = jnp.pad(input, ((0, NPAD - N), (0, 0)))
    hp = jnp.pad(hidden, ((0, NPAD - N), (0, 0)))

    t1 = _gconv(xp, hp, src, dst, w, zeros, W22)
    z, rh = _mid(t1, hidden, b2)

    rhp = jnp.pad(rh, ((0, NPAD - N), (0, 0)))
    t2 = _gconv(xp, rhp, src, dst, w, zeros, W22)
    output = _final(t2, z, hidden, b2)
    return (output, output)
